# baseline (speedup 1.0000x reference)
import jax
import jax.numpy as jnp
from jax.experimental import pallas as pl

EPS = 1e-6
BN_EPS = 1e-5
NEG_SLOPE = 0.2
K = 20


def _knn(x, k):
    inner = -2.0 * jnp.einsum('bcn,bcm->bnm', x, x)
    xx = jnp.sum(x * x, axis=1)
    pairwise = -xx[:, :, None] - inner - xx[:, None, :]
    _, idx = jax.lax.top_k(pairwise, k)
    return idx


def _get_graph_feature(x, k):
    B, nd, _, N = x.shape
    xf = x.reshape(B, nd * 3, N)
    idx = _knn(xf, k)
    xt = jnp.transpose(xf, (0, 2, 1))
    feature = jax.vmap(lambda xb, ib: xb[ib])(xt, idx)
    feature = feature.reshape(B, N, k, nd, 3)
    xc = xt.reshape(B, N, 1, nd, 3)
    xcb = jnp.broadcast_to(xc, (B, N, k, nd, 3))
    feature = jnp.concatenate([feature - xcb, xcb], axis=3)
    return jnp.transpose(feature, (0, 3, 4, 1, 2))


def _vn_batchnorm(p, gamma, beta):
    norm = jnp.sqrt(jnp.sum(p * p, axis=2)) + EPS
    axes = (0,) + tuple(range(2, norm.ndim))
    mean = jnp.mean(norm, axis=axes, keepdims=True)
    var = jnp.mean((norm - mean) ** 2, axis=axes, keepdims=True)
    shape = (1, gamma.shape[0]) + (1,) * (norm.ndim - 2)
    norm_bn = (norm - mean) / jnp.sqrt(var + BN_EPS) * gamma.reshape(shape) + beta.reshape(shape)
    return p / norm[:, :, None] * norm_bn[:, :, None]


def _vn_linear_leaky_relu(x, W_feat, W_dir, gamma, beta):
    p = jnp.einsum('oi,bi...->bo...', W_feat, x)
    p = _vn_batchnorm(p, gamma, beta)
    d = jnp.einsum('oi,bi...->bo...', W_dir, x)
    dotprod = jnp.sum(p * d, axis=2, keepdims=True)
    mask = (dotprod >= 0).astype(p.dtype)
    d_norm_sq = jnp.sum(d * d, axis=2, keepdims=True)
    return NEG_SLOPE * p + (1.0 - NEG_SLOPE) * (mask * p + (1.0 - mask) * (p - (dotprod / (d_norm_sq + EPS)) * d))


def _mean_last_pallas(x):
    B, C, three, N = x.shape
    xr = x.reshape(B * C * three, N)

    def body(x_ref, o_ref):
        o_ref[...] = jnp.mean(x_ref[...], axis=1, keepdims=True)

    out = pl.pallas_call(
        body,
        out_shape=jax.ShapeDtypeStruct((B * C * three, 1), x.dtype),
    )(xr)
    return out.reshape(B, C, three)


def kernel(x, W1, D1, g1, b1, W2, D2, g2, b2, W3, D3, g3, b3, W4, D4, g4, b4, Wc, Dc, gc, bc):
    xv = jnp.transpose(x[:, None, :, :], (0, 1, 3, 2))
    f = _get_graph_feature(xv, K)
    f = _vn_linear_leaky_relu(f, W1, D1, g1, b1)
    x1 = jnp.mean(f, axis=-1)
    f = _get_graph_feature(x1, K)
    f = _vn_linear_leaky_relu(f, W2, D2, g2, b2)
    x2 = jnp.mean(f, axis=-1)
    f = _get_graph_feature(x2, K)
    f = _vn_linear_leaky_relu(f, W3, D3, g3, b3)
    x3 = jnp.mean(f, axis=-1)
    f = _get_graph_feature(x3, K)
    f = _vn_linear_leaky_relu(f, W4, D4, g4, b4)
    x4 = jnp.mean(f, axis=-1)
    xc = jnp.concatenate([x1, x2, x3, x4], axis=1)
    out = _vn_linear_leaky_relu(xc, Wc, Dc, gc, bc)
    return _mean_last_pallas(out)


# speedup vs baseline: 4.2658x; 4.2658x over previous
import functools
import jax
import jax.numpy as jnp
from jax import lax
from jax.experimental import pallas as pl
from jax.experimental.pallas import tpu as pltpu
from jax.experimental.pallas import tpu_sc as plsc

EPS = 1e-6
BN_EPS = 1e-5
NEG_SLOPE = 0.2
K = 20
B, N = 4, 1024
BN_ = B * N
BNK = B * N * K
RB = 128
PA = 32
RG = PA * K


def _knn_body(x_ref, o_ref):
    b = pl.program_id(0)
    ib = pl.program_id(1)
    x = x_ref[...]
    rows = x_ref[pl.ds(ib * RB, RB), :]
    E = lax.dot_general(rows.astype(jnp.bfloat16), x.astype(jnp.bfloat16),
                        (((1,), (1,)), ((), ())),
                        preferred_element_type=jnp.float32)
    xx = jnp.sum(x * x, axis=1)
    xxr = jnp.sum(rows * rows, axis=1)
    dist = 2.0 * E - xxr[:, None]
    dist = dist - xx[None, :]
    iota = lax.broadcasted_iota(jnp.int32, (RB, N), 1)
    sel = []
    for _ in range(K):
        m = jnp.max(dist, axis=1, keepdims=True)
        cand = jnp.where(dist == m, iota, N)
        am = jnp.min(cand, axis=1, keepdims=True)
        sel.append(am)
        dist = jnp.where(iota == am, -jnp.inf, dist)
    o_ref[...] = jnp.concatenate(sel, axis=1) + b * N


def _knn_idx(X_nd):
    C = X_nd.shape[2]
    return pl.pallas_call(
        _knn_body,
        grid=(B, N // RB),
        in_specs=[pl.BlockSpec((None, N, C), lambda b, i: (b, 0, 0))],
        out_specs=pl.BlockSpec((None, RB, K), lambda b, i: (b, i, 0)),
        out_shape=jax.ShapeDtypeStruct((B, N, K), jnp.int32),
    )(X_nd)


def _sc_gather(table, gidx, Cg, chunk=256):
    info = plsc.get_sparse_core_info()
    NC, NS = info.num_cores, info.num_subcores
    NW = NC * NS
    per_w = BNK // NW
    nch = per_w // chunk
    mesh = plsc.VectorSubcoreMesh(core_axis_name="c", subcore_axis_name="s")

    @functools.partial(
        pl.kernel, mesh=mesh,
        out_type=jax.ShapeDtypeStruct((BNK, Cg), jnp.float32),
        scratch_types=[
            pltpu.VMEM((chunk,), jnp.int32),
            pltpu.VMEM((chunk, Cg), jnp.float32),
            pltpu.SemaphoreType.DMA,
        ],
    )
    def k(table_hbm, gidx_hbm, out_hbm, idx_v, rows_v, sem):
        wid = lax.axis_index("s") * NC + lax.axis_index("c")
        base = wid * per_w

        def body(i, carry):
            off = base + i * chunk
            pltpu.sync_copy(gidx_hbm.at[pl.ds(off, chunk)], idx_v)
            pltpu.async_copy(table_hbm.at[idx_v], rows_v, sem).wait()
            pltpu.sync_copy(rows_v, out_hbm.at[pl.ds(off, chunk)])
            return carry

        lax.fori_loop(0, nch, body, 0)

    return k(table, gidx)


def _passA_body(g_ref, c_ref, wt_ref, dt_ref, pd_ref, st_ref, *, nd):
    r = pl.program_id(0)
    Cf = 3 * nd
    g = g_ref[...]
    c = c_ref[...]
    g3 = g[:, :Cf].reshape(PA, K, Cf)
    fn = g3 - c[:, None, :]
    wt = wt_ref[...]
    dt = dt_ref[...]
    ps, ds = [], []
    for j in range(3):
        fnj = fn[:, :, j * nd:(j + 1) * nd]
        cj = jnp.broadcast_to(c[:, None, j * nd:(j + 1) * nd], (PA, K, nd))
        fcat = jnp.concatenate([fnj, cj], axis=2).reshape(RG, 2 * nd)
        fcat = fcat.astype(jnp.bfloat16)
        ps.append(lax.dot_general(fcat, wt, (((1,), (0,)), ((), ())),
                                  preferred_element_type=jnp.float32))
        ds.append(lax.dot_general(fcat, dt, (((1,), (0,)), ((), ())),
                                  preferred_element_type=jnp.float32))
    norm = jnp.sqrt(ps[0] * ps[0] + ps[1] * ps[1] + ps[2] * ps[2]) + EPS
    s1 = jnp.sum(norm, axis=0)
    s2 = jnp.sum(norm * norm, axis=0)
    st = jnp.stack([s1, s2], axis=0)

    @pl.when(r == 0)
    def _():
        st_ref[...] = jnp.zeros_like(st_ref)

    st_ref[...] += st
    pd_ref[...] = jnp.concatenate(ps + ds, axis=1)


def _passA(G, Xj, Wt, Dt, nd):
    Cg = G.shape[1]
    Cf = 3 * nd
    grid = (BNK // RG,)
    return pl.pallas_call(
        functools.partial(_passA_body, nd=nd),
        grid=grid,
        in_specs=[
            pl.BlockSpec((RG, Cg), lambda r: (r, 0)),
            pl.BlockSpec((PA, Cf), lambda r: (r, 0)),
            pl.BlockSpec((2 * nd, 64), lambda r: (0, 0)),
            pl.BlockSpec((2 * nd, 64), lambda r: (0, 0)),
        ],
        out_specs=[
            pl.BlockSpec((RG, 384), lambda r: (r, 0)),
            pl.BlockSpec((2, 64), lambda r: (0, 0)),
        ],
        out_shape=[
            jax.ShapeDtypeStruct((BNK, 384), jnp.float32),
            jax.ShapeDtypeStruct((2, 64), jnp.float32),
        ],
    )(G, Xj, Wt, Dt)


def _passB_body(pd_ref, st_ref, g_ref, b_ref, o_ref):
    pd = pd_ref[...]
    st = st_ref[...]
    gm = g_ref[...]
    bt = b_ref[...]
    Mf = jnp.float32(BNK)
    mu = st[0:1, :] / Mf
    var = st[1:2, :] / Mf - mu * mu
    p = [pd[:, j * 64:(j + 1) * 64] for j in range(3)]
    d = [pd[:, 192 + j * 64:192 + (j + 1) * 64] for j in range(3)]
    norm = jnp.sqrt(p[0] * p[0] + p[1] * p[1] + p[2] * p[2]) + EPS
    norm_bn = (norm - mu) / jnp.sqrt(var + BN_EPS) * gm + bt
    pbn = [(p[j] / norm) * norm_bn for j in range(3)]
    dot = pbn[0] * d[0] + pbn[1] * d[1] + pbn[2] * d[2]
    dsq = d[0] * d[0] + d[1] * d[1] + d[2] * d[2]
    mask = (dot >= 0).astype(jnp.float32)
    rr = dot / (dsq + EPS)
    outs = []
    for j in range(3):
        oj = NEG_SLOPE * pbn[j] + (1.0 - NEG_SLOPE) * (
            mask * pbn[j] + (1.0 - mask) * (pbn[j] - rr * d[j]))
        outs.append(jnp.mean(oj.reshape(PA, K, 64), axis=1))
    o_ref[...] = jnp.concatenate(outs, axis=1)


def _passB(PD, stats, g, b):
    grid = (BNK // RG,)
    return pl.pallas_call(
        _passB_body,
        grid=grid,
        in_specs=[
            pl.BlockSpec((RG, 384), lambda r: (r, 0)),
            pl.BlockSpec((2, 64), lambda r: (0, 0)),
            pl.BlockSpec((1, 64), lambda r: (0, 0)),
            pl.BlockSpec((1, 64), lambda r: (0, 0)),
        ],
        out_specs=pl.BlockSpec((PA, 192), lambda r: (r, 0)),
        out_shape=jax.ShapeDtypeStruct((BN_, 192), jnp.float32),
    )(PD, stats, g.reshape(1, 64), b.reshape(1, 64))


RF = 512


def _finalA_body(x1_ref, x2_ref, x3_ref, x4_ref, wt_ref, dt_ref, p_ref, dv_ref, st_ref):
    r = pl.program_id(0)
    xs = [x1_ref[...], x2_ref[...], x3_ref[...], x4_ref[...]]
    wt = wt_ref[...]
    dt = dt_ref[...]
    ps, dvs = [], []
    for j in range(3):
        xc = jnp.concatenate([x[:, j * 64:(j + 1) * 64] for x in xs], axis=1)
        xc = xc.astype(jnp.bfloat16)
        ps.append(lax.dot_general(xc, wt, (((1,), (0,)), ((), ())),
                                  preferred_element_type=jnp.float32))
        dvs.append(lax.dot_general(xc, dt, (((1,), (0,)), ((), ())),
                                   preferred_element_type=jnp.float32))
    norm = jnp.sqrt(ps[0] * ps[0] + ps[1] * ps[1] + ps[2] * ps[2]) + EPS
    s1 = jnp.sum(norm, axis=0)
    s2 = jnp.sum(norm * norm, axis=0)
    st = jnp.stack([s1, s2], axis=0)

    @pl.when(r == 0)
    def _():
        st_ref[...] = jnp.zeros_like(st_ref)

    st_ref[...] += st
    p_ref[...] = jnp.concatenate(ps, axis=1)
    dv_ref[...] = jnp.concatenate(dvs, axis=1)


def _finalB_body(p_ref, dv_ref, st_ref, g_ref, b_ref, o_ref):
    r = pl.program_id(0)
    pf = p_ref[...]
    dv = dv_ref[...]
    st = st_ref[...]
    gm = g_ref[...]
    bt = b_ref[...]
    Mf = jnp.float32(BN_)
    mu = st[0:1, :] / Mf
    var = st[1:2, :] / Mf - mu * mu
    p = [pf[:, j * 128:(j + 1) * 128] for j in range(3)]
    d = [dv[:, j * 8:j * 8 + 1] for j in range(3)]
    norm = jnp.sqrt(p[0] * p[0] + p[1] * p[1] + p[2] * p[2]) + EPS
    norm_bn = (norm - mu) / jnp.sqrt(var + BN_EPS) * gm + bt
    pbn = [(p[j] / norm) * norm_bn for j in range(3)]
    dot = pbn[0] * d[0] + pbn[1] * d[1] + pbn[2] * d[2]
    dsq = d[0] * d[0] + d[1] * d[1] + d[2] * d[2]
    mask = (dot >= 0).astype(jnp.float32)
    rr = dot / (dsq + EPS)
    outs = []
    for j in range(3):
        oj = NEG_SLOPE * pbn[j] + (1.0 - NEG_SLOPE) * (
            mask * pbn[j] + (1.0 - mask) * (pbn[j] - rr * d[j]))
        outs.append(jnp.sum(oj, axis=0, keepdims=True) * (1.0 / N))
    o = jnp.concatenate(outs, axis=0)

    nb = N // RF

    @pl.when(r % nb == 0)
    def _():
        o_ref[...] = jnp.zeros_like(o_ref)

    o_ref[...] += o


def _final(x1, x2, x3, x4, Wct, Dct, gc, bc):
    grid = (BN_ // RF,)
    nb = N // RF
    PF, DV, ST = pl.pallas_call(
        _finalA_body,
        grid=grid,
        in_specs=[pl.BlockSpec((RF, 192), lambda r: (r, 0))] * 4 + [
            pl.BlockSpec((256, 128), lambda r: (0, 0)),
            pl.BlockSpec((256, 8), lambda r: (0, 0)),
        ],
        out_specs=[
            pl.BlockSpec((RF, 384), lambda r: (r, 0)),
            pl.BlockSpec((RF, 24), lambda r: (r, 0)),
            pl.BlockSpec((2, 128), lambda r: (0, 0)),
        ],
        out_shape=[
            jax.ShapeDtypeStruct((BN_, 384), jnp.float32),
            jax.ShapeDtypeStruct((BN_, 24), jnp.float32),
            jax.ShapeDtypeStruct((2, 128), jnp.float32),
        ],
    )(x1, x2, x3, x4, Wct, Dct)
    out = pl.pallas_call(
        _finalB_body,
        grid=grid,
        in_specs=[
            pl.BlockSpec((RF, 384), lambda r: (r, 0)),
            pl.BlockSpec((RF, 24), lambda r: (r, 0)),
            pl.BlockSpec((2, 128), lambda r: (0, 0)),
            pl.BlockSpec((1, 128), lambda r: (0, 0)),
            pl.BlockSpec((1, 128), lambda r: (0, 0)),
        ],
        out_specs=pl.BlockSpec((None, 3, 128), lambda r: (r // nb, 0, 0)),
        out_shape=jax.ShapeDtypeStruct((B, 3, 128), jnp.float32),
    )(PF, DV, ST, gc.reshape(1, 128), bc.reshape(1, 128))
    return out


def _layer(X_nd, Xj, table, W, D, g, b, nd):
    idx = _knn_idx(X_nd).reshape(BNK)
    G = _sc_gather(table, idx, table.shape[1])
    Wt = W.T.astype(jnp.bfloat16)
    Dt = D.T.astype(jnp.bfloat16)
    PD, stats = _passA(G, Xj, Wt, Dt, nd)
    Xn_j = _passB(PD, stats, g, b)
    return Xn_j


def kernel(x, W1, D1, g1, b1, W2, D2, g2, b2, W3, D3, g3, b3, W4, D4, g4, b4, Wc, Dc, gc, bc):
    X1_nd = x
    X1j = x.reshape(BN_, 3)
    t1 = jnp.pad(X1j, ((0, 0), (0, 125)))
    x1 = _layer(X1_nd, X1j, t1, W1, D1, g1, b1, 1)

    def prep(Xj):
        X_nd = Xj.reshape(BN_, 3, 64).transpose(0, 2, 1).reshape(B, N, 192)
        tab = jnp.pad(Xj, ((0, 0), (0, 64)))
        return X_nd, tab

    X_nd, tab = prep(x1)
    x2 = _layer(X_nd, x1, tab, W2, D2, g2, b2, 64)
    X_nd, tab = prep(x2)
    x3 = _layer(X_nd, x2, tab, W3, D3, g3, b3, 64)
    X_nd, tab = prep(x3)
    x4 = _layer(X_nd, x3, tab, W4, D4, g4, b4, 64)

    Wct = Wc.T.astype(jnp.bfloat16)
    Dct = jnp.pad(Dc.T, ((0, 0), (0, 7))).astype(jnp.bfloat16)
    out = _final(x1, x2, x3, x4, Wct, Dct, gc, bc)
    return jnp.transpose(out, (0, 2, 1))


# speedup vs baseline: 4.4151x; 1.0350x over previous
import functools
import jax
import jax.numpy as jnp
from jax import lax
from jax.experimental import pallas as pl
from jax.experimental.pallas import tpu as pltpu
from jax.experimental.pallas import tpu_sc as plsc

EPS = 1e-6
BN_EPS = 1e-5
NEG_SLOPE = 0.2
K = 20
B, N = 4, 1024
BN_ = B * N
BNK = B * N * K
RB = 128
PA = 32
RG = PA * K


def _knn_body(x_ref, o_ref):
    b = pl.program_id(0)
    ib = pl.program_id(1)
    x = x_ref[...]
    rows = x_ref[pl.ds(ib * RB, RB), :]
    E = lax.dot_general(rows.astype(jnp.bfloat16), x.astype(jnp.bfloat16),
                        (((1,), (1,)), ((), ())),
                        preferred_element_type=jnp.float32)
    xx = jnp.sum(x * x, axis=1)
    xxr = jnp.sum(rows * rows, axis=1)
    dist = 2.0 * E - xxr[:, None]
    dist = dist - xx[None, :]
    iota = lax.broadcasted_iota(jnp.int32, (RB, N), 1)
    sel = []
    for _ in range(K):
        m = jnp.max(dist, axis=1, keepdims=True)
        cand = jnp.where(dist == m, iota, N)
        am = jnp.min(cand, axis=1, keepdims=True)
        sel.append(am)
        dist = jnp.where(iota == am, -jnp.inf, dist)
    o_ref[...] = jnp.concatenate(sel, axis=1) + b * N


def _knn_idx(X_nd):
    C = X_nd.shape[2]
    return pl.pallas_call(
        _knn_body,
        grid=(B, N // RB),
        in_specs=[pl.BlockSpec((None, N, C), lambda b, i: (b, 0, 0))],
        out_specs=pl.BlockSpec((None, RB, K), lambda b, i: (b, i, 0)),
        out_shape=jax.ShapeDtypeStruct((B, N, K), jnp.int32),
    )(X_nd)


def _sc_gather(table, gidx, Cg, chunk=160):
    info = plsc.get_sparse_core_info()
    NC, NS = info.num_cores, info.num_subcores
    NW = NC * NS
    per_w = BNK // NW
    nch = per_w // chunk
    mesh = plsc.VectorSubcoreMesh(core_axis_name="c", subcore_axis_name="s")

    @functools.partial(
        pl.kernel, mesh=mesh,
        out_type=jax.ShapeDtypeStruct((BNK, Cg), jnp.float32),
        scratch_types=[
            pltpu.VMEM((per_w,), jnp.int32),
            pltpu.VMEM((chunk, Cg), jnp.float32),
            pltpu.VMEM((chunk, Cg), jnp.float32),
            pltpu.SemaphoreType.DMA,
            pltpu.SemaphoreType.DMA,
        ],
    )
    def k(table_hbm, gidx_hbm, out_hbm, idx_v, rows0_v, rows1_v, sem0, sem1):
        wid = lax.axis_index("s") * NC + lax.axis_index("c")
        base = wid * per_w
        pltpu.sync_copy(gidx_hbm.at[pl.ds(base, per_w)], idx_v)
        rows = (rows0_v, rows1_v)
        sems = (sem0, sem1)

        pltpu.async_copy(table_hbm.at[idx_v.at[pl.ds(0, chunk)]], rows0_v, sem0)

        def body(i, carry):
            for par in range(2):
                @pl.when(i % 2 == par)
                def _():
                    @pl.when(i + 1 < nch)
                    def _():
                        pltpu.async_copy(
                            table_hbm.at[idx_v.at[pl.ds((i + 1) * chunk, chunk)]],
                            rows[1 - par], sems[1 - par])
                    pltpu.make_async_copy(table_hbm.at[idx_v.at[pl.ds(i * chunk, chunk)]],
                                          rows[par], sems[par]).wait()
                    pltpu.sync_copy(rows[par], out_hbm.at[pl.ds(base + i * chunk, chunk)])
            return carry

        lax.fori_loop(0, nch, body, 0)

    return k(table, gidx)


def _passA_body(g_ref, c_ref, wdt_ref, pd_ref, st_ref, *, nd):
    r = pl.program_id(0)
    Cf = 3 * nd
    g = g_ref[...]
    c = c_ref[...]
    crep = jnp.repeat(c, K, axis=0)
    fn = g[:, :Cf] - crep
    wdt = wdt_ref[...]
    fcats = []
    for j in range(3):
        fnj = fn[:, j * nd:(j + 1) * nd]
        cj = crep[:, j * nd:(j + 1) * nd]
        fcats.append(jnp.concatenate([fnj, cj], axis=1))
    fcat = jnp.concatenate(fcats, axis=0).astype(jnp.bfloat16)
    pdj = lax.dot_general(fcat, wdt, (((1,), (0,)), ((), ())),
                          preferred_element_type=jnp.float32)
    ps = [pdj[j * RG:(j + 1) * RG, :64] for j in range(3)]
    ds = [pdj[j * RG:(j + 1) * RG, 64:] for j in range(3)]
    norm = jnp.sqrt(ps[0] * ps[0] + ps[1] * ps[1] + ps[2] * ps[2]) + EPS
    s1 = jnp.sum(norm, axis=0)
    s2 = jnp.sum(norm * norm, axis=0)
    st = jnp.stack([s1, s2], axis=0)

    @pl.when(r == 0)
    def _():
        st_ref[...] = jnp.zeros_like(st_ref)

    st_ref[...] += st
    pd_ref[...] = jnp.concatenate(ps + ds, axis=1)


def _passA(G, Xj, WDt, nd):
    Cg = G.shape[1]
    Cf = 3 * nd
    grid = (BNK // RG,)
    return pl.pallas_call(
        functools.partial(_passA_body, nd=nd),
        grid=grid,
        in_specs=[
            pl.BlockSpec((RG, Cg), lambda r: (r, 0)),
            pl.BlockSpec((PA, Cf), lambda r: (r, 0)),
            pl.BlockSpec((2 * nd, 128), lambda r: (0, 0)),
        ],
        out_specs=[
            pl.BlockSpec((RG, 384), lambda r: (r, 0)),
            pl.BlockSpec((2, 64), lambda r: (0, 0)),
        ],
        out_shape=[
            jax.ShapeDtypeStruct((BNK, 384), jnp.float32),
            jax.ShapeDtypeStruct((2, 64), jnp.float32),
        ],
    )(G, Xj, WDt)


def _passB_body(pd_ref, st_ref, g_ref, b_ref, o_ref):
    pd = pd_ref[...]
    st = st_ref[...]
    gm = g_ref[...]
    bt = b_ref[...]
    Mf = jnp.float32(BNK)
    mu = st[0:1, :] / Mf
    var = st[1:2, :] / Mf - mu * mu
    p = [pd[:, j * 64:(j + 1) * 64] for j in range(3)]
    d = [pd[:, 192 + j * 64:192 + (j + 1) * 64] for j in range(3)]
    norm = jnp.sqrt(p[0] * p[0] + p[1] * p[1] + p[2] * p[2]) + EPS
    norm_bn = (norm - mu) / jnp.sqrt(var + BN_EPS) * gm + bt
    pbn = [(p[j] / norm) * norm_bn for j in range(3)]
    dot = pbn[0] * d[0] + pbn[1] * d[1] + pbn[2] * d[2]
    dsq = d[0] * d[0] + d[1] * d[1] + d[2] * d[2]
    mask = (dot >= 0).astype(jnp.float32)
    rr = dot / (dsq + EPS)
    outs = []
    for j in range(3):
        oj = NEG_SLOPE * pbn[j] + (1.0 - NEG_SLOPE) * (
            mask * pbn[j] + (1.0 - mask) * (pbn[j] - rr * d[j]))
        outs.append(jnp.mean(oj.reshape(PA, K, 64), axis=1))
    o_ref[...] = jnp.concatenate(outs, axis=1)


def _passB(PD, stats, g, b):
    grid = (BNK // RG,)
    return pl.pallas_call(
        _passB_body,
        grid=grid,
        in_specs=[
            pl.BlockSpec((RG, 384), lambda r: (r, 0)),
            pl.BlockSpec((2, 64), lambda r: (0, 0)),
            pl.BlockSpec((1, 64), lambda r: (0, 0)),
            pl.BlockSpec((1, 64), lambda r: (0, 0)),
        ],
        out_specs=pl.BlockSpec((PA, 192), lambda r: (r, 0)),
        out_shape=jax.ShapeDtypeStruct((BN_, 192), jnp.float32),
    )(PD, stats, g.reshape(1, 64), b.reshape(1, 64))


RF = 512


def _finalA_body(x1_ref, x2_ref, x3_ref, x4_ref, wt_ref, dt_ref, p_ref, dv_ref, st_ref):
    r = pl.program_id(0)
    xs = [x1_ref[...], x2_ref[...], x3_ref[...], x4_ref[...]]
    wt = wt_ref[...]
    dt = dt_ref[...]
    ps, dvs = [], []
    for j in range(3):
        xc = jnp.concatenate([x[:, j * 64:(j + 1) * 64] for x in xs], axis=1)
        xc = xc.astype(jnp.bfloat16)
        ps.append(lax.dot_general(xc, wt, (((1,), (0,)), ((), ())),
                                  preferred_element_type=jnp.float32))
        dvs.append(lax.dot_general(xc, dt, (((1,), (0,)), ((), ())),
                                   preferred_element_type=jnp.float32))
    norm = jnp.sqrt(ps[0] * ps[0] + ps[1] * ps[1] + ps[2] * ps[2]) + EPS
    s1 = jnp.sum(norm, axis=0)
    s2 = jnp.sum(norm * norm, axis=0)
    st = jnp.stack([s1, s2], axis=0)

    @pl.when(r == 0)
    def _():
        st_ref[...] = jnp.zeros_like(st_ref)

    st_ref[...] += st
    p_ref[...] = jnp.concatenate(ps, axis=1)
    dv_ref[...] = jnp.concatenate(dvs, axis=1)


def _finalB_body(p_ref, dv_ref, st_ref, g_ref, b_ref, o_ref):
    r = pl.program_id(0)
    pf = p_ref[...]
    dv = dv_ref[...]
    st = st_ref[...]
    gm = g_ref[...]
    bt = b_ref[...]
    Mf = jnp.float32(BN_)
    mu = st[0:1, :] / Mf
    var = st[1:2, :] / Mf - mu * mu
    p = [pf[:, j * 128:(j + 1) * 128] for j in range(3)]
    d = [dv[:, j * 8:j * 8 + 1] for j in range(3)]
    norm = jnp.sqrt(p[0] * p[0] + p[1] * p[1] + p[2] * p[2]) + EPS
    norm_bn = (norm - mu) / jnp.sqrt(var + BN_EPS) * gm + bt
    pbn = [(p[j] / norm) * norm_bn for j in range(3)]
    dot = pbn[0] * d[0] + pbn[1] * d[1] + pbn[2] * d[2]
    dsq = d[0] * d[0] + d[1] * d[1] + d[2] * d[2]
    mask = (dot >= 0).astype(jnp.float32)
    rr = dot / (dsq + EPS)
    outs = []
    for j in range(3):
        oj = NEG_SLOPE * pbn[j] + (1.0 - NEG_SLOPE) * (
            mask * pbn[j] + (1.0 - mask) * (pbn[j] - rr * d[j]))
        outs.append(jnp.sum(oj, axis=0, keepdims=True) * (1.0 / N))
    o = jnp.concatenate(outs, axis=0)

    nb = N // RF

    @pl.when(r % nb == 0)
    def _():
        o_ref[...] = jnp.zeros_like(o_ref)

    o_ref[...] += o


def _final(x1, x2, x3, x4, Wct, Dct, gc, bc):
    grid = (BN_ // RF,)
    nb = N // RF
    PF, DV, ST = pl.pallas_call(
        _finalA_body,
        grid=grid,
        in_specs=[pl.BlockSpec((RF, 192), lambda r: (r, 0))] * 4 + [
            pl.BlockSpec((256, 128), lambda r: (0, 0)),
            pl.BlockSpec((256, 8), lambda r: (0, 0)),
        ],
        out_specs=[
            pl.BlockSpec((RF, 384), lambda r: (r, 0)),
            pl.BlockSpec((RF, 24), lambda r: (r, 0)),
            pl.BlockSpec((2, 128), lambda r: (0, 0)),
        ],
        out_shape=[
            jax.ShapeDtypeStruct((BN_, 384), jnp.float32),
            jax.ShapeDtypeStruct((BN_, 24), jnp.float32),
            jax.ShapeDtypeStruct((2, 128), jnp.float32),
        ],
    )(x1, x2, x3, x4, Wct, Dct)
    out = pl.pallas_call(
        _finalB_body,
        grid=grid,
        in_specs=[
            pl.BlockSpec((RF, 384), lambda r: (r, 0)),
            pl.BlockSpec((RF, 24), lambda r: (r, 0)),
            pl.BlockSpec((2, 128), lambda r: (0, 0)),
            pl.BlockSpec((1, 128), lambda r: (0, 0)),
            pl.BlockSpec((1, 128), lambda r: (0, 0)),
        ],
        out_specs=pl.BlockSpec((None, 3, 128), lambda r: (r // nb, 0, 0)),
        out_shape=jax.ShapeDtypeStruct((B, 3, 128), jnp.float32),
    )(PF, DV, ST, gc.reshape(1, 128), bc.reshape(1, 128))
    return out


def _layer(X_nd, Xj, table, W, D, g, b, nd):
    idx = _knn_idx(X_nd).reshape(BNK)
    G = _sc_gather(table, idx, table.shape[1])
    WDt = jnp.concatenate([W.T, D.T], axis=1).astype(jnp.bfloat16)
    PD, stats = _passA(G, Xj, WDt, nd)
    Xn_j = _passB(PD, stats, g, b)
    return Xn_j


def kernel(x, W1, D1, g1, b1, W2, D2, g2, b2, W3, D3, g3, b3, W4, D4, g4, b4, Wc, Dc, gc, bc):
    X1_nd = x
    X1j = x.reshape(BN_, 3)
    t1 = jnp.pad(X1j, ((0, 0), (0, 125)))
    x1 = _layer(X1_nd, X1j, t1, W1, D1, g1, b1, 1)

    def prep(Xj):
        X_nd = Xj.reshape(BN_, 3, 64).transpose(0, 2, 1).reshape(B, N, 192)
        tab = jnp.pad(Xj, ((0, 0), (0, 64)))
        return X_nd, tab

    X_nd, tab = prep(x1)
    x2 = _layer(X_nd, x1, tab, W2, D2, g2, b2, 64)
    X_nd, tab = prep(x2)
    x3 = _layer(X_nd, x2, tab, W3, D3, g3, b3, 64)
    X_nd, tab = prep(x3)
    x4 = _layer(X_nd, x3, tab, W4, D4, g4, b4, 64)

    Wct = Wc.T.astype(jnp.bfloat16)
    Dct = jnp.pad(Dc.T, ((0, 0), (0, 7))).astype(jnp.bfloat16)
    out = _final(x1, x2, x3, x4, Wct, Dct, gc, bc)
    return jnp.transpose(out, (0, 2, 1))
